# initial kernel scaffold (unmeasured)
import jax
import jax.numpy as jnp
from jax import lax
from jax.experimental import pallas as pl
from jax.experimental.pallas import tpu as pltpu

W = 4


def kernel(x, w_mat):
    M, Ks = x.shape
    N = w_mat.shape[1]
    MC = M // W

    def body(x_ref, w_ref, out_ref,
             rs_send, rs_recv, ag_buf, amax_buf,
             rs_send_sems, rs_recv_sems,
             ag_send_sems, ag_recv_sems,
             amax_send_sems, amax_recv_sems):
        my = lax.axis_index("i")
        right = (my + 1) % W
        left = (my - 1) % W

        barrier_sem = pltpu.get_barrier_semaphore()
        for nbr in (left, right):
            pl.semaphore_signal(
                barrier_sem, inc=1,
                device_id=(nbr,), device_id_type=pl.DeviceIdType.MESH,
            )
        pl.semaphore_wait(barrier_sem, 2)

        w_bf = w_ref[...].astype(jnp.bfloat16)

        def partial_chunk(c):
            xc = x_ref[pl.ds(c * MC, MC), :].astype(jnp.bfloat16)
            return lax.dot_general(
                xc, w_bf, (((1,), (0,)), ((), ())),
                preferred_element_type=jnp.float32,
            )

        rs_send[0, ...] = partial_chunk((my - 1) % W).astype(jnp.bfloat16)
        final = None
        for s in range(W - 1):
            send_slot = s % 2
            rdma = pltpu.make_async_remote_copy(
                src_ref=rs_send.at[send_slot],
                dst_ref=rs_recv.at[s],
                send_sem=rs_send_sems.at[send_slot],
                recv_sem=rs_recv_sems.at[s],
                device_id=(right,),
                device_id_type=pl.DeviceIdType.MESH,
            )
            rdma.start()
            p_next = partial_chunk((my - 2 - s) % W)
            rdma.wait()
            acc = rs_recv[s, ...].astype(jnp.float32) + p_next
            if s < W - 2:
                rs_send[(s + 1) % 2, ...] = acc.astype(jnp.bfloat16)
            else:
                final = acc

        r = jnp.maximum(final, 0.0)
        amax_local = jnp.max(r)
        amax_buf[pl.ds(my, 1), ...] = jnp.full(
            (1, 8, 128), amax_local, jnp.float32)

        sends = []
        for o in range(1, W):
            peer = (my + o) % W
            snd = pltpu.make_async_remote_copy(
                src_ref=amax_buf.at[my],
                dst_ref=amax_buf.at[my],
                send_sem=amax_send_sems.at[o - 1],
                recv_sem=amax_recv_sems.at[my],
                device_id=(peer,),
                device_id_type=pl.DeviceIdType.MESH,
            )
            snd.start()
            sends.append(snd)
        for snd in sends:
            snd.wait_send()
        for o in range(1, W):
            peer = (my + o) % W
            rcv = pltpu.make_async_remote_copy(
                src_ref=amax_buf.at[peer],
                dst_ref=amax_buf.at[peer],
                send_sem=amax_send_sems.at[0],
                recv_sem=amax_recv_sems.at[peer],
                device_id=(peer,),
                device_id_type=pl.DeviceIdType.MESH,
            )
            rcv.wait_recv()

        amax_g = jnp.max(amax_buf[...])
        scale = jnp.maximum(amax_g / 127.0, 1e-30)

        q = jnp.clip(jnp.round(r / scale), 0.0, 127.0)
        ag_buf[pl.ds(my, 1), ...] = q.astype(jnp.int8).reshape(1, MC, N)
        out_ref[pl.ds(my * MC, MC), :] = q * scale

        for h in range(W - 1):
            sc = (my - h) % W
            rc = (my - h - 1) % W
            rdma = pltpu.make_async_remote_copy(
                src_ref=ag_buf.at[sc],
                dst_ref=ag_buf.at[sc],
                send_sem=ag_send_sems.at[h],
                recv_sem=ag_recv_sems.at[h],
                device_id=(right,),
                device_id_type=pl.DeviceIdType.MESH,
            )
            rdma.start()
            rdma.wait()
            out_ref[pl.ds(rc * MC, MC), :] = (
                ag_buf[rc, ...].astype(jnp.float32) * scale)

    return pl.pallas_call(
        body,
        out_shape=jax.ShapeDtypeStruct((M, N), jnp.float32),
        in_specs=[
            pl.BlockSpec(memory_space=pltpu.VMEM),
            pl.BlockSpec(memory_space=pltpu.VMEM),
        ],
        out_specs=pl.BlockSpec(memory_space=pltpu.VMEM),
        scratch_shapes=[
            pltpu.VMEM((2, MC, N), jnp.bfloat16),
            pltpu.VMEM((W - 1, MC, N), jnp.bfloat16),
            pltpu.VMEM((W, MC, N), jnp.int8),
            pltpu.VMEM((W, 8, 128), jnp.float32),
            pltpu.SemaphoreType.DMA((2,)),
            pltpu.SemaphoreType.DMA((W - 1,)),
            pltpu.SemaphoreType.DMA((W - 1,)),
            pltpu.SemaphoreType.DMA((W - 1,)),
            pltpu.SemaphoreType.DMA((W - 1,)),
            pltpu.SemaphoreType.DMA((W,)),
        ],
        compiler_params=pltpu.CompilerParams(collective_id=0),
    )(x, w_mat)


# baseline (device time: 288354 ns/iter reference)
import jax
import jax.numpy as jnp
from jax import lax
from jax.experimental import pallas as pl
from jax.experimental.pallas import tpu as pltpu

W = 4
TR = 128


def kernel(x, w_mat):
    M, Ks = x.shape
    N = w_mat.shape[1]
    MC = M // W
    NT = MC // TR

    def body(x_ref, w_ref, out_ref,
             rs_send, rs_recv, ag_buf, amax_buf, stage,
             rs_send_sems, rs_recv_sems,
             ag_send_sems, ag_recv_sems,
             amax_send_sems, amax_recv_sems,
             stage_sems):
        my = lax.axis_index("i")
        right = (my + 1) % W
        left = (my - 1) % W

        barrier_sem = pltpu.get_barrier_semaphore()
        for nbr in (left, right):
            pl.semaphore_signal(
                barrier_sem, inc=1,
                device_id=(nbr,), device_id_type=pl.DeviceIdType.MESH,
            )
        pl.semaphore_wait(barrier_sem, 2)

        def partial_into(slot, c):
            for t in range(NT):
                p = lax.dot_general(
                    x_ref[pl.ds(c * MC + t * TR, TR), :], w_ref[...],
                    (((1,), (0,)), ((), ())),
                    preferred_element_type=jnp.float32,
                )
                rs_send[slot, pl.ds(t * TR, TR), :] = p.astype(jnp.bfloat16)

        def add_recv(slot, hop):
            for t in range(NT):
                rows = pl.ds(t * TR, TR)
                s = (rs_send[slot, rows, :].astype(jnp.float32)
                     + rs_recv[hop, rows, :].astype(jnp.float32))
                rs_send[slot, rows, :] = s.astype(jnp.bfloat16)

        partial_into(0, (my - 1) % W)
        for s in range(W - 1):
            rdma = pltpu.make_async_remote_copy(
                src_ref=rs_send.at[s % 2],
                dst_ref=rs_recv.at[s],
                send_sem=rs_send_sems.at[s % 2],
                recv_sem=rs_recv_sems.at[s],
                device_id=(right,),
                device_id_type=pl.DeviceIdType.MESH,
            )
            rdma.start()
            partial_into((s + 1) % 2, (my - 2 - s) % W)
            rdma.wait()
            add_recv((s + 1) % 2, s)

        amax_local = jnp.float32(0.0)
        for t in range(NT):
            rows = pl.ds(t * TR, TR)
            r = jnp.maximum(rs_send[1, rows, :].astype(jnp.float32), 0.0)
            amax_local = jnp.maximum(amax_local, jnp.max(r))
            rs_send[1, rows, :] = r.astype(jnp.bfloat16)
        amax_buf[pl.ds(my, 1), ...] = jnp.full(
            (1, 8, 128), amax_local, jnp.float32)

        sends = []
        for o in range(1, W):
            peer = (my + o) % W
            snd = pltpu.make_async_remote_copy(
                src_ref=amax_buf.at[my],
                dst_ref=amax_buf.at[my],
                send_sem=amax_send_sems.at[o - 1],
                recv_sem=amax_recv_sems.at[my],
                device_id=(peer,),
                device_id_type=pl.DeviceIdType.MESH,
            )
            snd.start()
            sends.append(snd)
        for snd in sends:
            snd.wait_send()
        for o in range(1, W):
            peer = (my + o) % W
            rcv = pltpu.make_async_remote_copy(
                src_ref=amax_buf.at[peer],
                dst_ref=amax_buf.at[peer],
                send_sem=amax_send_sems.at[0],
                recv_sem=amax_recv_sems.at[peer],
                device_id=(peer,),
                device_id_type=pl.DeviceIdType.MESH,
            )
            rcv.wait_recv()

        amax_g = jnp.max(amax_buf[...])
        scale = jnp.maximum(amax_g / 127.0, 1e-30)

        pending = [None, None]

        def stage_tile(values_f32, out_row):
            t_slot = stage_tile.counter % 2
            stage_tile.counter += 1
            if pending[t_slot] is not None:
                pending[t_slot].wait()
            stage[t_slot, ...] = values_f32
            cp = pltpu.make_async_copy(
                stage.at[t_slot],
                out_ref.at[pl.ds(out_row, TR), :],
                stage_sems.at[t_slot],
            )
            cp.start()
            pending[t_slot] = cp
        stage_tile.counter = 0

        for t in range(NT):
            rows = pl.ds(t * TR, TR)
            qf = jnp.clip(
                jnp.round(rs_send[1, rows, :].astype(jnp.float32) / scale),
                0.0, 127.0)
            ag_buf[pl.ds(my, 1), rows, :] = (
                qf.astype(jnp.int8).reshape(1, TR, N))
            stage_tile(qf * scale, my * MC + t * TR)

        for h in range(W - 1):
            sc = (my - h) % W
            rc = (my - h - 1) % W
            rdma = pltpu.make_async_remote_copy(
                src_ref=ag_buf.at[sc],
                dst_ref=ag_buf.at[sc],
                send_sem=ag_send_sems.at[h],
                recv_sem=ag_recv_sems.at[h],
                device_id=(right,),
                device_id_type=pl.DeviceIdType.MESH,
            )
            rdma.start()
            rdma.wait()
            for t in range(NT):
                rows = pl.ds(t * TR, TR)
                stage_tile(
                    ag_buf[rc, rows, :].astype(jnp.float32) * scale,
                    rc * MC + t * TR)
        for cp in pending:
            if cp is not None:
                cp.wait()

    return pl.pallas_call(
        body,
        out_shape=jax.ShapeDtypeStruct((M, N), jnp.float32),
        in_specs=[
            pl.BlockSpec(memory_space=pltpu.VMEM),
            pl.BlockSpec(memory_space=pltpu.VMEM),
        ],
        out_specs=pl.BlockSpec(memory_space=pl.ANY),
        scratch_shapes=[
            pltpu.VMEM((2, MC, N), jnp.bfloat16),
            pltpu.VMEM((W - 1, MC, N), jnp.bfloat16),
            pltpu.VMEM((W, MC, N), jnp.int8),
            pltpu.VMEM((W, 8, 128), jnp.float32),
            pltpu.VMEM((2, TR, N), jnp.float32),
            pltpu.SemaphoreType.DMA((2,)),
            pltpu.SemaphoreType.DMA((W - 1,)),
            pltpu.SemaphoreType.DMA((W - 1,)),
            pltpu.SemaphoreType.DMA((W - 1,)),
            pltpu.SemaphoreType.DMA((W - 1,)),
            pltpu.SemaphoreType.DMA((W,)),
            pltpu.SemaphoreType.DMA((2,)),
        ],
        compiler_params=pltpu.CompilerParams(
            collective_id=0,
            vmem_limit_bytes=50 * 1024 * 1024,
        ),
    )(x.astype(jnp.bfloat16), w_mat.astype(jnp.bfloat16))


# device time: 195478 ns/iter; 1.4751x vs baseline; 1.4751x over previous
import jax
import jax.numpy as jnp
from jax import lax
from jax.experimental import pallas as pl
from jax.experimental.pallas import tpu as pltpu

W = 4
TR = 128


def kernel(x, w_mat):
    M, Ks = x.shape
    N = w_mat.shape[1]
    MC = M // W
    NH = N // 2
    NT = MC // TR

    def body(x_ref, w_ref, out_ref,
             rs_send_r, rs_send_l, rs_recv_r, rs_recv_l,
             ag_r, ag_l, amax_buf, stage,
             rs_send_sems_r, rs_send_sems_l,
             rs_recv_sems_r, rs_recv_sems_l,
             ag_send_sems_r, ag_send_sems_l,
             ag_recv_sems_r, ag_recv_sems_l,
             amax_send_sems, amax_recv_sems,
             stage_sems):
        my = lax.axis_index("i")
        right = (my + 1) % W
        left = (my - 1) % W

        barrier_sem = pltpu.get_barrier_semaphore()
        for nbr in (left, right):
            pl.semaphore_signal(
                barrier_sem, inc=1,
                device_id=(nbr,), device_id_type=pl.DeviceIdType.MESH,
            )
        pl.semaphore_wait(barrier_sem, 2)

        def partial_into(dst, slot, c, col0):
            for t in range(NT):
                p = lax.dot_general(
                    x_ref[pl.ds(c * MC + t * TR, TR), :],
                    w_ref[:, pl.ds(col0, NH)],
                    (((1,), (0,)), ((), ())),
                    preferred_element_type=jnp.float32,
                )
                dst[slot, pl.ds(t * TR, TR), :] = p.astype(jnp.bfloat16)

        def add_recv(dst, slot, recv, hop):
            for t in range(NT):
                rows = pl.ds(t * TR, TR)
                s = (dst[slot, rows, :].astype(jnp.float32)
                     + recv[hop, rows, :].astype(jnp.float32))
                dst[slot, rows, :] = s.astype(jnp.bfloat16)

        partial_into(rs_send_r, 0, (my - 1) % W, 0)
        partial_into(rs_send_l, 0, (my + 1) % W, NH)
        for s in range(W - 1):
            rdma_r = pltpu.make_async_remote_copy(
                src_ref=rs_send_r.at[s % 2],
                dst_ref=rs_recv_r.at[s],
                send_sem=rs_send_sems_r.at[s % 2],
                recv_sem=rs_recv_sems_r.at[s],
                device_id=(right,),
                device_id_type=pl.DeviceIdType.MESH,
            )
            rdma_l = pltpu.make_async_remote_copy(
                src_ref=rs_send_l.at[s % 2],
                dst_ref=rs_recv_l.at[s],
                send_sem=rs_send_sems_l.at[s % 2],
                recv_sem=rs_recv_sems_l.at[s],
                device_id=(left,),
                device_id_type=pl.DeviceIdType.MESH,
            )
            rdma_r.start()
            rdma_l.start()
            partial_into(rs_send_r, (s + 1) % 2, (my - 2 - s) % W, 0)
            partial_into(rs_send_l, (s + 1) % 2, (my + 2 + s) % W, NH)
            rdma_r.wait()
            add_recv(rs_send_r, (s + 1) % 2, rs_recv_r, s)
            rdma_l.wait()
            add_recv(rs_send_l, (s + 1) % 2, rs_recv_l, s)

        amax_local = jnp.float32(0.0)
        for half in (rs_send_r, rs_send_l):
            for t in range(NT):
                rows = pl.ds(t * TR, TR)
                r = jnp.maximum(half[1, rows, :].astype(jnp.float32), 0.0)
                amax_local = jnp.maximum(amax_local, jnp.max(r))
                half[1, rows, :] = r.astype(jnp.bfloat16)
        amax_buf[pl.ds(my, 1), ...] = jnp.full(
            (1, 8, 128), amax_local, jnp.float32)

        sends = []
        for o in range(1, W):
            peer = (my + o) % W
            snd = pltpu.make_async_remote_copy(
                src_ref=amax_buf.at[my],
                dst_ref=amax_buf.at[my],
                send_sem=amax_send_sems.at[o - 1],
                recv_sem=amax_recv_sems.at[my],
                device_id=(peer,),
                device_id_type=pl.DeviceIdType.MESH,
            )
            snd.start()
            sends.append(snd)
        for snd in sends:
            snd.wait_send()
        for o in range(1, W):
            peer = (my + o) % W
            rcv = pltpu.make_async_remote_copy(
                src_ref=amax_buf.at[peer],
                dst_ref=amax_buf.at[peer],
                send_sem=amax_send_sems.at[0],
                recv_sem=amax_recv_sems.at[peer],
                device_id=(peer,),
                device_id_type=pl.DeviceIdType.MESH,
            )
            rcv.wait_recv()

        amax_g = jnp.max(amax_buf[...])
        scale = jnp.maximum(amax_g / 127.0, 1e-30)

        pending = [None, None]

        def stage_tile(values_f32, out_row, col0):
            t_slot = stage_tile.counter % 2
            stage_tile.counter += 1
            if pending[t_slot] is not None:
                pending[t_slot].wait()
            stage[t_slot, :, pl.ds(col0, NH)] = values_f32
            cp = pltpu.make_async_copy(
                stage.at[t_slot, :, pl.ds(col0, NH)],
                out_ref.at[pl.ds(out_row, TR), pl.ds(col0, NH)],
                stage_sems.at[t_slot],
            )
            cp.start()
            pending[t_slot] = cp
        stage_tile.counter = 0

        for half, ag, col0 in ((rs_send_r, ag_r, 0), (rs_send_l, ag_l, NH)):
            for t in range(NT):
                rows = pl.ds(t * TR, TR)
                qf = jnp.clip(
                    jnp.round(half[1, rows, :].astype(jnp.float32) / scale),
                    0.0, 127.0)
                ag[pl.ds(my, 1), rows, :] = (
                    qf.astype(jnp.int8).reshape(1, TR, NH))
                stage_tile(qf * scale, my * MC + t * TR, col0)

        for h in range(W - 1):
            sc_r = (my - h) % W
            rc_r = (my - h - 1) % W
            sc_l = (my + h) % W
            rc_l = (my + h + 1) % W
            rdma_r = pltpu.make_async_remote_copy(
                src_ref=ag_r.at[sc_r],
                dst_ref=ag_r.at[sc_r],
                send_sem=ag_send_sems_r.at[h],
                recv_sem=ag_recv_sems_r.at[h],
                device_id=(right,),
                device_id_type=pl.DeviceIdType.MESH,
            )
            rdma_l = pltpu.make_async_remote_copy(
                src_ref=ag_l.at[sc_l],
                dst_ref=ag_l.at[sc_l],
                send_sem=ag_send_sems_l.at[h],
                recv_sem=ag_recv_sems_l.at[h],
                device_id=(left,),
                device_id_type=pl.DeviceIdType.MESH,
            )
            rdma_r.start()
            rdma_l.start()
            rdma_r.wait()
            for t in range(NT):
                stage_tile(
                    ag_r[rc_r, pl.ds(t * TR, TR), :].astype(jnp.float32)
                    * scale,
                    rc_r * MC + t * TR, 0)
            rdma_l.wait()
            for t in range(NT):
                stage_tile(
                    ag_l[rc_l, pl.ds(t * TR, TR), :].astype(jnp.float32)
                    * scale,
                    rc_l * MC + t * TR, NH)
        for cp in pending:
            if cp is not None:
                cp.wait()

    return pl.pallas_call(
        body,
        out_shape=jax.ShapeDtypeStruct((M, N), jnp.float32),
        in_specs=[
            pl.BlockSpec(memory_space=pltpu.VMEM),
            pl.BlockSpec(memory_space=pltpu.VMEM),
        ],
        out_specs=pl.BlockSpec(memory_space=pl.ANY),
        scratch_shapes=[
            pltpu.VMEM((2, MC, NH), jnp.bfloat16),
            pltpu.VMEM((2, MC, NH), jnp.bfloat16),
            pltpu.VMEM((W - 1, MC, NH), jnp.bfloat16),
            pltpu.VMEM((W - 1, MC, NH), jnp.bfloat16),
            pltpu.VMEM((W, MC, NH), jnp.int8),
            pltpu.VMEM((W, MC, NH), jnp.int8),
            pltpu.VMEM((W, 8, 128), jnp.float32),
            pltpu.VMEM((2, TR, N), jnp.float32),
            pltpu.SemaphoreType.DMA((2,)),
            pltpu.SemaphoreType.DMA((2,)),
            pltpu.SemaphoreType.DMA((W - 1,)),
            pltpu.SemaphoreType.DMA((W - 1,)),
            pltpu.SemaphoreType.DMA((W - 1,)),
            pltpu.SemaphoreType.DMA((W - 1,)),
            pltpu.SemaphoreType.DMA((W - 1,)),
            pltpu.SemaphoreType.DMA((W - 1,)),
            pltpu.SemaphoreType.DMA((W - 1,)),
            pltpu.SemaphoreType.DMA((W,)),
            pltpu.SemaphoreType.DMA((2,)),
        ],
        compiler_params=pltpu.CompilerParams(
            collective_id=0,
            vmem_limit_bytes=50 * 1024 * 1024,
        ),
    )(x.astype(jnp.bfloat16), w_mat.astype(jnp.bfloat16))


# device time: 157837 ns/iter; 1.8269x vs baseline; 1.2385x over previous
import jax
import jax.numpy as jnp
from jax import lax
from jax.experimental import pallas as pl
from jax.experimental.pallas import tpu as pltpu

W = 4
NS = 4


def kernel(x, w_mat):
    M, Ks = x.shape
    N = w_mat.shape[1]
    MC = M // W
    NH = N // 2
    SB = MC // NS

    def body(x_ref, w_ref, out_ref,
             rs_send_r, rs_send_l, rs_recv_r, rs_recv_l,
             ag_r, ag_l, amax_buf, stage,
             rs_send_sems_r, rs_send_sems_l,
             rs_recv_sems_r, rs_recv_sems_l,
             ag_send_sems_r, ag_send_sems_l,
             ag_recv_sems_r, ag_recv_sems_l,
             amax_send_sems, amax_recv_sems,
             stage_sems):
        my = lax.axis_index("i")
        right = (my + 1) % W
        left = (my - 1) % W

        DIRS = (
            dict(nbr=right, col0=0, rs_chunk=lambda s: (my - 1 - s) % W,
                 send=rs_send_r, recv=rs_recv_r, ag=ag_r,
                 send_sems=rs_send_sems_r, recv_sems=rs_recv_sems_r,
                 ag_send_sems=ag_send_sems_r, ag_recv_sems=ag_recv_sems_r,
                 ag_sc=lambda h: (my - h) % W, ag_rc=lambda h: (my - h - 1) % W),
            dict(nbr=left, col0=NH, rs_chunk=lambda s: (my + 1 + s) % W,
                 send=rs_send_l, recv=rs_recv_l, ag=ag_l,
                 send_sems=rs_send_sems_l, recv_sems=rs_recv_sems_l,
                 ag_send_sems=ag_send_sems_l, ag_recv_sems=ag_recv_sems_l,
                 ag_sc=lambda h: (my + h) % W, ag_rc=lambda h: (my + h + 1) % W),
        )

        barrier_sem = pltpu.get_barrier_semaphore()
        for nbr in (left, right):
            pl.semaphore_signal(
                barrier_sem, inc=1,
                device_id=(nbr,), device_id_type=pl.DeviceIdType.MESH,
            )
        pl.semaphore_wait(barrier_sem, 2)

        def rowsub(k):
            return pl.ds(k * SB, SB)

        def dot_sub(c, k, col0):
            return lax.dot_general(
                x_ref[pl.ds(c * MC + k * SB, SB), :],
                w_ref[:, pl.ds(col0, NH)],
                (((1,), (0,)), ((), ())),
                preferred_element_type=jnp.float32,
            )

        def rs_rdma(d, s, k):
            return pltpu.make_async_remote_copy(
                src_ref=d["send"].at[s, rowsub(k), :],
                dst_ref=d["recv"].at[s, rowsub(k), :],
                send_sem=d["send_sems"].at[s * NS + k],
                recv_sem=d["recv_sems"].at[s * NS + k],
                device_id=(d["nbr"],),
                device_id_type=pl.DeviceIdType.MESH,
            )

        def ag_rdma(d, h, k):
            sc = d["ag_sc"](h)
            return pltpu.make_async_remote_copy(
                src_ref=d["ag"].at[sc, rowsub(k), :],
                dst_ref=d["ag"].at[sc, rowsub(k), :],
                send_sem=d["ag_send_sems"].at[h * NS + k],
                recv_sem=d["ag_recv_sems"].at[h * NS + k],
                device_id=(d["nbr"],),
                device_id_type=pl.DeviceIdType.MESH,
            )

        for k in range(NS):
            for d in DIRS:
                p = dot_sub(d["rs_chunk"](0), k, d["col0"])
                d["send"][0, rowsub(k), :] = p.astype(jnp.bfloat16)
                rs_rdma(d, 0, k).start()

        for s in range(1, W - 1):
            for k in range(NS):
                for d in DIRS:
                    p = dot_sub(d["rs_chunk"](s), k, d["col0"])
                    rs_rdma(d, s - 1, k).wait_recv()
                    acc = p + d["recv"][s - 1, rowsub(k), :].astype(jnp.float32)
                    d["send"][s, rowsub(k), :] = acc.astype(jnp.bfloat16)
                    rs_rdma(d, s, k).start()

        amax_local = jnp.float32(0.0)
        for k in range(NS):
            for d in DIRS:
                p = dot_sub(my, k, d["col0"])
                rs_rdma(d, W - 2, k).wait_recv()
                r = jnp.maximum(
                    p + d["recv"][W - 2, rowsub(k), :].astype(jnp.float32),
                    0.0)
                amax_local = jnp.maximum(amax_local, jnp.max(r))
                d["recv"][W - 2, rowsub(k), :] = r.astype(jnp.bfloat16)
        amax_buf[pl.ds(my, 1), ...] = jnp.full(
            (1, 8, 128), amax_local, jnp.float32)

        amax_sends = []
        for o in range(1, W):
            peer = (my + o) % W
            snd = pltpu.make_async_remote_copy(
                src_ref=amax_buf.at[my],
                dst_ref=amax_buf.at[my],
                send_sem=amax_send_sems.at[o - 1],
                recv_sem=amax_recv_sems.at[my],
                device_id=(peer,),
                device_id_type=pl.DeviceIdType.MESH,
            )
            snd.start()
            amax_sends.append(snd)
        for o in range(1, W):
            peer = (my + o) % W
            rcv = pltpu.make_async_remote_copy(
                src_ref=amax_buf.at[peer],
                dst_ref=amax_buf.at[peer],
                send_sem=amax_send_sems.at[0],
                recv_sem=amax_recv_sems.at[peer],
                device_id=(peer,),
                device_id_type=pl.DeviceIdType.MESH,
            )
            rcv.wait_recv()

        amax_g = jnp.max(amax_buf[...])
        scale = jnp.maximum(amax_g / 127.0, 1e-30)

        pending = [None, None]

        def stage_sub(values_f32, out_row, col0):
            t_slot = stage_sub.counter % 2
            stage_sub.counter += 1
            if pending[t_slot] is not None:
                pending[t_slot].wait()
            stage[t_slot, :, pl.ds(col0, NH)] = values_f32
            cp = pltpu.make_async_copy(
                stage.at[t_slot, :, pl.ds(col0, NH)],
                out_ref.at[pl.ds(out_row, SB), pl.ds(col0, NH)],
                stage_sems.at[t_slot],
            )
            cp.start()
            pending[t_slot] = cp
        stage_sub.counter = 0

        for k in range(NS):
            for d in DIRS:
                qf = jnp.clip(
                    jnp.round(
                        d["recv"][W - 2, rowsub(k), :].astype(jnp.float32)
                        / scale),
                    0.0, 127.0)
                d["ag"][pl.ds(my, 1), rowsub(k), :] = (
                    qf.astype(jnp.int8).reshape(1, SB, NH))
                ag_rdma(d, 0, k).start()
                stage_sub(qf * scale, my * MC + k * SB, d["col0"])

        for h in range(W - 1):
            for k in range(NS):
                for d in DIRS:
                    rc = d["ag_rc"](h)
                    ag_rdma(d, h, k).wait_recv()
                    if h < W - 2:
                        ag_rdma(d, h + 1, k).start()
                    stage_sub(
                        d["ag"][rc, rowsub(k), :].astype(jnp.float32) * scale,
                        rc * MC + k * SB, d["col0"])

        for s in range(W - 1):
            for k in range(NS):
                for d in DIRS:
                    rs_rdma(d, s, k).wait_send()
                    ag_rdma(d, s, k).wait_send()
        for snd in amax_sends:
            snd.wait_send()
        for cp in pending:
            if cp is not None:
                cp.wait()

    nsub = (W - 1) * NS
    return pl.pallas_call(
        body,
        out_shape=jax.ShapeDtypeStruct((M, N), jnp.float32),
        in_specs=[
            pl.BlockSpec(memory_space=pltpu.VMEM),
            pl.BlockSpec(memory_space=pltpu.VMEM),
        ],
        out_specs=pl.BlockSpec(memory_space=pl.ANY),
        scratch_shapes=[
            pltpu.VMEM((W - 1, MC, NH), jnp.bfloat16),
            pltpu.VMEM((W - 1, MC, NH), jnp.bfloat16),
            pltpu.VMEM((W - 1, MC, NH), jnp.bfloat16),
            pltpu.VMEM((W - 1, MC, NH), jnp.bfloat16),
            pltpu.VMEM((W, MC, NH), jnp.int8),
            pltpu.VMEM((W, MC, NH), jnp.int8),
            pltpu.VMEM((W, 8, 128), jnp.float32),
            pltpu.VMEM((2, SB, N), jnp.float32),
            pltpu.SemaphoreType.DMA((nsub,)),
            pltpu.SemaphoreType.DMA((nsub,)),
            pltpu.SemaphoreType.DMA((nsub,)),
            pltpu.SemaphoreType.DMA((nsub,)),
            pltpu.SemaphoreType.DMA((nsub,)),
            pltpu.SemaphoreType.DMA((nsub,)),
            pltpu.SemaphoreType.DMA((nsub,)),
            pltpu.SemaphoreType.DMA((nsub,)),
            pltpu.SemaphoreType.DMA((W - 1,)),
            pltpu.SemaphoreType.DMA((W,)),
            pltpu.SemaphoreType.DMA((2,)),
        ],
        compiler_params=pltpu.CompilerParams(
            collective_id=0,
            vmem_limit_bytes=51 * 1024 * 1024,
        ),
    )(x.astype(jnp.bfloat16), w_mat.astype(jnp.bfloat16))


# device time: 157750 ns/iter; 1.8279x vs baseline; 1.0006x over previous
import jax
import jax.numpy as jnp
from jax import lax
from jax.experimental import pallas as pl
from jax.experimental.pallas import tpu as pltpu

W = 4
NS = 4


def kernel(x, w_mat):
    M, Ks = x.shape
    N = w_mat.shape[1]
    MC = M // W
    NH = N // 2
    SB = MC // NS

    def body(x_ref, w_ref, out_ref,
             rs_send_r, rs_send_l, rs_recv_r, rs_recv_l,
             ag_r, ag_l, amax_buf, stage,
             rs_send_sems_r, rs_send_sems_l,
             rs_recv_sems_r, rs_recv_sems_l,
             ag_send_sems_r, ag_send_sems_l,
             ag_recv_sems_r, ag_recv_sems_l,
             amax_send_sems, amax_recv_sems,
             stage_sems):
        my = lax.axis_index("i")
        right = (my + 1) % W
        left = (my - 1) % W

        DIRS = (
            dict(nbr=right, col0=0, rs_chunk=lambda s: (my - 1 - s) % W,
                 send=rs_send_r, recv=rs_recv_r, ag=ag_r,
                 send_sems=rs_send_sems_r, recv_sems=rs_recv_sems_r,
                 ag_send_sems=ag_send_sems_r, ag_recv_sems=ag_recv_sems_r,
                 ag_sc=lambda h: (my - h) % W, ag_rc=lambda h: (my - h - 1) % W),
            dict(nbr=left, col0=NH, rs_chunk=lambda s: (my + 1 + s) % W,
                 send=rs_send_l, recv=rs_recv_l, ag=ag_l,
                 send_sems=rs_send_sems_l, recv_sems=rs_recv_sems_l,
                 ag_send_sems=ag_send_sems_l, ag_recv_sems=ag_recv_sems_l,
                 ag_sc=lambda h: (my + h) % W, ag_rc=lambda h: (my + h + 1) % W),
        )

        barrier_sem = pltpu.get_barrier_semaphore()
        for nbr in (left, right):
            pl.semaphore_signal(
                barrier_sem, inc=1,
                device_id=(nbr,), device_id_type=pl.DeviceIdType.MESH,
            )
        pl.semaphore_wait(barrier_sem, 2)

        def rowsub(k):
            return pl.ds(k * SB, SB)

        def dot_sub(c, k, col0):
            return lax.dot_general(
                x_ref[pl.ds(c * MC + k * SB, SB), :],
                w_ref[:, pl.ds(col0, NH)],
                (((1,), (0,)), ((), ())),
                preferred_element_type=jnp.float32,
            )

        def rs_rdma(d, s, k):
            return pltpu.make_async_remote_copy(
                src_ref=d["send"].at[s, rowsub(k), :],
                dst_ref=d["recv"].at[s, rowsub(k), :],
                send_sem=d["send_sems"].at[s * NS + k],
                recv_sem=d["recv_sems"].at[s * NS + k],
                device_id=(d["nbr"],),
                device_id_type=pl.DeviceIdType.MESH,
            )

        def ag_rdma(d, h, k):
            sc = d["ag_sc"](h)
            return pltpu.make_async_remote_copy(
                src_ref=d["ag"].at[sc, rowsub(k), :],
                dst_ref=d["ag"].at[sc, rowsub(k), :],
                send_sem=d["ag_send_sems"].at[h * NS + k],
                recv_sem=d["ag_recv_sems"].at[h * NS + k],
                device_id=(d["nbr"],),
                device_id_type=pl.DeviceIdType.MESH,
            )

        for k in range(NS):
            for d in DIRS:
                p = dot_sub(d["rs_chunk"](0), k, d["col0"])
                d["send"][0, rowsub(k), :] = p.astype(jnp.bfloat16)
                rs_rdma(d, 0, k).start()

        for s in range(1, W - 1):
            for k in range(NS):
                for d in DIRS:
                    p = dot_sub(d["rs_chunk"](s), k, d["col0"])
                    rs_rdma(d, s - 1, k).wait_recv()
                    acc = p + d["recv"][s - 1, rowsub(k), :].astype(jnp.float32)
                    d["send"][s, rowsub(k), :] = acc.astype(jnp.bfloat16)
                    rs_rdma(d, s, k).start()

        amax_local = jnp.float32(0.0)
        for k in range(NS):
            for d in DIRS:
                p = dot_sub(my, k, d["col0"])
                rs_rdma(d, W - 2, k).wait_recv()
                r = jnp.maximum(
                    p + d["recv"][W - 2, rowsub(k), :].astype(jnp.float32),
                    0.0)
                amax_local = jnp.maximum(amax_local, jnp.max(r))
                d["recv"][W - 2, rowsub(k), :] = r.astype(jnp.bfloat16)
        amax_buf[pl.ds(my, 1), ...] = jnp.full(
            (1, 8, 128), amax_local, jnp.float32)

        amax_sends = []
        for o in range(1, W):
            peer = (my + o) % W
            snd = pltpu.make_async_remote_copy(
                src_ref=amax_buf.at[my],
                dst_ref=amax_buf.at[my],
                send_sem=amax_send_sems.at[o - 1],
                recv_sem=amax_recv_sems.at[my],
                device_id=(peer,),
                device_id_type=pl.DeviceIdType.MESH,
            )
            snd.start()
            amax_sends.append(snd)
        for o in range(1, W):
            peer = (my + o) % W
            rcv = pltpu.make_async_remote_copy(
                src_ref=amax_buf.at[peer],
                dst_ref=amax_buf.at[peer],
                send_sem=amax_send_sems.at[0],
                recv_sem=amax_recv_sems.at[peer],
                device_id=(peer,),
                device_id_type=pl.DeviceIdType.MESH,
            )
            rcv.wait_recv()

        amax_g = jnp.max(amax_buf[...])
        scale = jnp.maximum(amax_g / 127.0, 1e-30)

        pending = [None, None]

        def stage_sub(values_f32, out_row, col0):
            t_slot = stage_sub.counter % 2
            stage_sub.counter += 1
            if pending[t_slot] is not None:
                pending[t_slot].wait()
            stage[t_slot, :, pl.ds(col0, NH)] = values_f32
            cp = pltpu.make_async_copy(
                stage.at[t_slot, :, pl.ds(col0, NH)],
                out_ref.at[pl.ds(out_row, SB), pl.ds(col0, NH)],
                stage_sems.at[t_slot],
            )
            cp.start()
            pending[t_slot] = cp
        stage_sub.counter = 0

        for k in range(NS):
            for d in DIRS:
                qf = jnp.clip(
                    jnp.round(
                        d["recv"][W - 2, rowsub(k), :].astype(jnp.float32)
                        / scale),
                    0.0, 127.0)
                d["ag"][pl.ds(my, 1), rowsub(k), :] = (
                    qf.astype(jnp.int8).reshape(1, SB, NH))
                ag_rdma(d, 0, k).start()
                stage_sub(qf * scale, my * MC + k * SB, d["col0"])

        for h in range(W - 1):
            for k in range(NS):
                for d in DIRS:
                    rc = d["ag_rc"](h)
                    ag_rdma(d, h, k).wait_recv()
                    if h < W - 2:
                        ag_rdma(d, h + 1, k).start()
                    stage_sub(
                        d["ag"][rc, rowsub(k), :].astype(jnp.float32) * scale,
                        rc * MC + k * SB, d["col0"])

        for s in range(W - 1):
            for k in range(NS):
                for d in DIRS:
                    rs_rdma(d, s, k).wait_send()
                    ag_rdma(d, s, k).wait_send()
        for snd in amax_sends:
            snd.wait_send()
        for cp in pending:
            if cp is not None:
                cp.wait()

    nsub = (W - 1) * NS
    return pl.pallas_call(
        body,
        out_shape=jax.ShapeDtypeStruct((M, N), jnp.float32),
        in_specs=[
            pl.BlockSpec(memory_space=pltpu.VMEM),
            pl.BlockSpec(memory_space=pltpu.VMEM),
        ],
        out_specs=pl.BlockSpec(memory_space=pltpu.MemorySpace.HBM),
        scratch_shapes=[
            pltpu.VMEM((W - 1, MC, NH), jnp.bfloat16),
            pltpu.VMEM((W - 1, MC, NH), jnp.bfloat16),
            pltpu.VMEM((W - 1, MC, NH), jnp.bfloat16),
            pltpu.VMEM((W - 1, MC, NH), jnp.bfloat16),
            pltpu.VMEM((W, MC, NH), jnp.int8),
            pltpu.VMEM((W, MC, NH), jnp.int8),
            pltpu.VMEM((W, 8, 128), jnp.float32),
            pltpu.VMEM((2, SB, N), jnp.float32),
            pltpu.SemaphoreType.DMA((nsub,)),
            pltpu.SemaphoreType.DMA((nsub,)),
            pltpu.SemaphoreType.DMA((nsub,)),
            pltpu.SemaphoreType.DMA((nsub,)),
            pltpu.SemaphoreType.DMA((nsub,)),
            pltpu.SemaphoreType.DMA((nsub,)),
            pltpu.SemaphoreType.DMA((nsub,)),
            pltpu.SemaphoreType.DMA((nsub,)),
            pltpu.SemaphoreType.DMA((W - 1,)),
            pltpu.SemaphoreType.DMA((W,)),
            pltpu.SemaphoreType.DMA((2,)),
        ],
        compiler_params=pltpu.CompilerParams(
            collective_id=0,
            vmem_limit_bytes=51 * 1024 * 1024,
        ),
    )(x.astype(jnp.bfloat16), w_mat.astype(jnp.bfloat16))


# device time: 136998 ns/iter; 2.1048x vs baseline; 1.1515x over previous
import jax
import jax.numpy as jnp
from jax import lax
from jax.experimental import pallas as pl
from jax.experimental.pallas import tpu as pltpu

W = 4
NS = 4


def kernel(x, w_mat):
    M, Ks = x.shape
    N = w_mat.shape[1]
    MC = M // W
    NH = N // 2
    SB = MC // NS

    def body(x_ref, w_ref, out_ref,
             rs_send_r, rs_send_l, rs_recv_r, rs_recv_l,
             ag_r, ag_l, amax_buf, stage, x_stage,
             rs_send_sems_r, rs_send_sems_l,
             rs_recv_sems_r, rs_recv_sems_l,
             ag_send_sems_r, ag_send_sems_l,
             ag_recv_sems_r, ag_recv_sems_l,
             amax_send_sems, amax_recv_sems,
             stage_sems, x_sems):
        my = lax.axis_index("i")
        right = (my + 1) % W
        left = (my - 1) % W

        DIRS = (
            dict(nbr=right, col0=0, rs_chunk=lambda s: (my - 1 - s) % W,
                 send=rs_send_r, recv=rs_recv_r, ag=ag_r,
                 send_sems=rs_send_sems_r, recv_sems=rs_recv_sems_r,
                 ag_send_sems=ag_send_sems_r, ag_recv_sems=ag_recv_sems_r,
                 ag_sc=lambda h: (my - h) % W, ag_rc=lambda h: (my - h - 1) % W),
            dict(nbr=left, col0=NH, rs_chunk=lambda s: (my + 1 + s) % W,
                 send=rs_send_l, recv=rs_recv_l, ag=ag_l,
                 send_sems=rs_send_sems_l, recv_sems=rs_recv_sems_l,
                 ag_send_sems=ag_send_sems_l, ag_recv_sems=ag_recv_sems_l,
                 ag_sc=lambda h: (my + h) % W, ag_rc=lambda h: (my + h + 1) % W),
        )

        def rowsub(k):
            return pl.ds(k * SB, SB)

        xtasks = []
        for k in range(NS):
            xtasks.append(((my - 1) % W, k))
            xtasks.append(((my + 1) % W, k))
        for k in range(NS):
            xtasks.append(((my + 2) % W, k))
        for k in range(NS):
            xtasks.append(((my + 1) % W, k))
            xtasks.append(((my - 1) % W, k))
        for k in range(NS):
            xtasks.append((my, k))

        def x_dma(i):
            c, k = xtasks[i]
            return pltpu.make_async_copy(
                x_ref.at[pl.ds(c * MC + k * SB, SB), :],
                x_stage.at[i % 2],
                x_sems.at[i % 2],
            )

        x_dma(0).start()
        x_dma(1).start()
        xcursor = [0]

        def next_x():
            i = xcursor[0]
            xcursor[0] = i + 1
            x_dma(i).wait()
            xv = x_stage[i % 2, ...].astype(jnp.bfloat16)
            if i + 2 < len(xtasks):
                x_dma(i + 2).start()
            return xv

        def dot_half(xv, col0):
            return lax.dot_general(
                xv, w_ref[:, pl.ds(col0, NH)],
                (((1,), (0,)), ((), ())),
                preferred_element_type=jnp.float32,
            )

        barrier_sem = pltpu.get_barrier_semaphore()
        for nbr in (left, right):
            pl.semaphore_signal(
                barrier_sem, inc=1,
                device_id=(nbr,), device_id_type=pl.DeviceIdType.MESH,
            )
        pl.semaphore_wait(barrier_sem, 2)

        def rs_rdma(d, s, k):
            return pltpu.make_async_remote_copy(
                src_ref=d["send"].at[s, rowsub(k), :],
                dst_ref=d["recv"].at[s, rowsub(k), :],
                send_sem=d["send_sems"].at[s * NS + k],
                recv_sem=d["recv_sems"].at[s * NS + k],
                device_id=(d["nbr"],),
                device_id_type=pl.DeviceIdType.MESH,
            )

        def ag_rdma(d, h, k):
            sc = d["ag_sc"](h)
            return pltpu.make_async_remote_copy(
                src_ref=d["ag"].at[sc, rowsub(k), :],
                dst_ref=d["ag"].at[sc, rowsub(k), :],
                send_sem=d["ag_send_sems"].at[h * NS + k],
                recv_sem=d["ag_recv_sems"].at[h * NS + k],
                device_id=(d["nbr"],),
                device_id_type=pl.DeviceIdType.MESH,
            )

        for k in range(NS):
            for d in DIRS:
                p = dot_half(next_x(), d["col0"])
                d["send"][0, rowsub(k), :] = p.astype(jnp.bfloat16)
                rs_rdma(d, 0, k).start()

        for k in range(NS):
            xv = next_x()
            for d in DIRS:
                p = dot_half(xv, d["col0"])
                rs_rdma(d, 0, k).wait_recv()
                acc = p + d["recv"][0, rowsub(k), :].astype(jnp.float32)
                d["send"][1, rowsub(k), :] = acc.astype(jnp.bfloat16)
                rs_rdma(d, 1, k).start()

        for k in range(NS):
            for d in DIRS:
                p = dot_half(next_x(), d["col0"])
                rs_rdma(d, 1, k).wait_recv()
                acc = p + d["recv"][1, rowsub(k), :].astype(jnp.float32)
                d["send"][2, rowsub(k), :] = acc.astype(jnp.bfloat16)
                rs_rdma(d, 2, k).start()

        amax_local = jnp.float32(0.0)
        for k in range(NS):
            xv = next_x()
            for d in DIRS:
                p = dot_half(xv, d["col0"])
                rs_rdma(d, W - 2, k).wait_recv()
                r = jnp.maximum(
                    p + d["recv"][W - 2, rowsub(k), :].astype(jnp.float32),
                    0.0)
                amax_local = jnp.maximum(amax_local, jnp.max(r))
                d["recv"][W - 2, rowsub(k), :] = r.astype(jnp.bfloat16)
        amax_buf[pl.ds(my, 1), ...] = jnp.full(
            (1, 8, 128), amax_local, jnp.float32)

        amax_sends = []
        for o in range(1, W):
            peer = (my + o) % W
            snd = pltpu.make_async_remote_copy(
                src_ref=amax_buf.at[my],
                dst_ref=amax_buf.at[my],
                send_sem=amax_send_sems.at[o - 1],
                recv_sem=amax_recv_sems.at[my],
                device_id=(peer,),
                device_id_type=pl.DeviceIdType.MESH,
            )
            snd.start()
            amax_sends.append(snd)
        for o in range(1, W):
            peer = (my + o) % W
            rcv = pltpu.make_async_remote_copy(
                src_ref=amax_buf.at[peer],
                dst_ref=amax_buf.at[peer],
                send_sem=amax_send_sems.at[0],
                recv_sem=amax_recv_sems.at[peer],
                device_id=(peer,),
                device_id_type=pl.DeviceIdType.MESH,
            )
            rcv.wait_recv()

        amax_g = jnp.max(amax_buf[...])
        scale = jnp.maximum(amax_g / 127.0, 1e-30)

        pending = [None, None]

        def stage_sub(values_f32, out_row, col0):
            t_slot = stage_sub.counter % 2
            stage_sub.counter += 1
            if pending[t_slot] is not None:
                pending[t_slot].wait()
            stage[t_slot, :, pl.ds(col0, NH)] = values_f32.astype(jnp.bfloat16)
            cp = pltpu.make_async_copy(
                stage.at[t_slot, :, pl.ds(col0, NH)],
                out_ref.at[pl.ds(out_row, SB), pl.ds(col0, NH)],
                stage_sems.at[t_slot],
            )
            cp.start()
            pending[t_slot] = cp
        stage_sub.counter = 0

        for k in range(NS):
            for d in DIRS:
                qf = jnp.clip(
                    jnp.round(
                        d["recv"][W - 2, rowsub(k), :].astype(jnp.float32)
                        / scale),
                    0.0, 127.0)
                d["ag"][pl.ds(my, 1), rowsub(k), :] = (
                    qf.astype(jnp.int8).reshape(1, SB, NH))
                ag_rdma(d, 0, k).start()
                stage_sub(qf * scale, my * MC + k * SB, d["col0"])

        for h in range(W - 1):
            for k in range(NS):
                for d in DIRS:
                    rc = d["ag_rc"](h)
                    ag_rdma(d, h, k).wait_recv()
                    if h < W - 2:
                        ag_rdma(d, h + 1, k).start()
                    stage_sub(
                        d["ag"][rc, rowsub(k), :].astype(jnp.float32) * scale,
                        rc * MC + k * SB, d["col0"])

        for s in range(W - 1):
            for k in range(NS):
                for d in DIRS:
                    rs_rdma(d, s, k).wait_send()
                    ag_rdma(d, s, k).wait_send()
        for snd in amax_sends:
            snd.wait_send()
        for cp in pending:
            if cp is not None:
                cp.wait()

    nsub = (W - 1) * NS
    return pl.pallas_call(
        body,
        out_shape=jax.ShapeDtypeStruct((M, N), jnp.bfloat16),
        in_specs=[
            pl.BlockSpec(memory_space=pl.ANY),
            pl.BlockSpec(memory_space=pltpu.MemorySpace.VMEM),
        ],
        out_specs=pl.BlockSpec(memory_space=pltpu.MemorySpace.HBM),
        scratch_shapes=[
            pltpu.VMEM((W - 1, MC, NH), jnp.bfloat16),
            pltpu.VMEM((W - 1, MC, NH), jnp.bfloat16),
            pltpu.VMEM((W - 1, MC, NH), jnp.bfloat16),
            pltpu.VMEM((W - 1, MC, NH), jnp.bfloat16),
            pltpu.VMEM((W, MC, NH), jnp.int8),
            pltpu.VMEM((W, MC, NH), jnp.int8),
            pltpu.VMEM((W, 8, 128), jnp.float32),
            pltpu.VMEM((2, SB, N), jnp.bfloat16),
            pltpu.VMEM((2, SB, Ks), jnp.float32),
            pltpu.SemaphoreType.DMA((nsub,)),
            pltpu.SemaphoreType.DMA((nsub,)),
            pltpu.SemaphoreType.DMA((nsub,)),
            pltpu.SemaphoreType.DMA((nsub,)),
            pltpu.SemaphoreType.DMA((nsub,)),
            pltpu.SemaphoreType.DMA((nsub,)),
            pltpu.SemaphoreType.DMA((nsub,)),
            pltpu.SemaphoreType.DMA((nsub,)),
            pltpu.SemaphoreType.DMA((W - 1,)),
            pltpu.SemaphoreType.DMA((W,)),
            pltpu.SemaphoreType.DMA((2,)),
            pltpu.SemaphoreType.DMA((2,)),
        ],
        compiler_params=pltpu.CompilerParams(
            collective_id=0,
            vmem_limit_bytes=51 * 1024 * 1024,
        ),
    )(x, w_mat.astype(jnp.bfloat16))


# device time: 132720 ns/iter; 2.1726x vs baseline; 1.0322x over previous
import jax
import jax.numpy as jnp
from jax import lax
from jax.experimental import pallas as pl
from jax.experimental.pallas import tpu as pltpu

W = 4
NS = 4


def kernel(x, w_mat):
    M, Ks = x.shape
    N = w_mat.shape[1]
    MC = M // W
    NH = N // 2
    SB = MC // NS

    def body(x_ref, w_ref, out_ref,
             rs_send_r, rs_send_l, rs_recv_r, rs_recv_l,
             ag_r, ag_l, amax_buf, stage, x_stage, w_stage, w_bf,
             rs_send_sems_r, rs_send_sems_l,
             rs_recv_sems_r, rs_recv_sems_l,
             ag_send_sems_r, ag_send_sems_l,
             ag_recv_sems_r, ag_recv_sems_l,
             amax_send_sems, amax_recv_sems,
             stage_sems, x_sems, w_sems):
        my = lax.axis_index("i")
        right = (my + 1) % W
        left = (my - 1) % W

        DIRS = (
            dict(nbr=right, col0=0, rs_chunk=lambda s: (my - 1 - s) % W,
                 send=rs_send_r, recv=rs_recv_r, ag=ag_r,
                 send_sems=rs_send_sems_r, recv_sems=rs_recv_sems_r,
                 ag_send_sems=ag_send_sems_r, ag_recv_sems=ag_recv_sems_r,
                 ag_sc=lambda h: (my - h) % W, ag_rc=lambda h: (my - h - 1) % W),
            dict(nbr=left, col0=NH, rs_chunk=lambda s: (my + 1 + s) % W,
                 send=rs_send_l, recv=rs_recv_l, ag=ag_l,
                 send_sems=rs_send_sems_l, recv_sems=rs_recv_sems_l,
                 ag_send_sems=ag_send_sems_l, ag_recv_sems=ag_recv_sems_l,
                 ag_sc=lambda h: (my + h) % W, ag_rc=lambda h: (my + h + 1) % W),
        )

        def rowsub(k):
            return pl.ds(k * SB, SB)

        xtasks = []
        for k in range(NS):
            xtasks.append(((my - 1) % W, k))
            xtasks.append(((my + 1) % W, k))
        for k in range(NS):
            xtasks.append(((my + 2) % W, k))
        for k in range(NS):
            xtasks.append(((my + 1) % W, k))
            xtasks.append(((my - 1) % W, k))
        for k in range(NS):
            xtasks.append((my, k))

        def x_dma(i):
            c, k = xtasks[i]
            return pltpu.make_async_copy(
                x_ref.at[pl.ds(c * MC + k * SB, SB), :],
                x_stage.at[i % 2],
                x_sems.at[i % 2],
            )

        w_dmas = [
            pltpu.make_async_copy(
                w_ref.at[:, pl.ds(h * NH, NH)], w_stage.at[h], w_sems.at[h])
            for h in range(2)
        ]
        for wd in w_dmas:
            wd.start()
        x_dma(0).start()
        x_dma(1).start()
        for h in range(2):
            w_dmas[h].wait()
            w_bf[:, pl.ds(h * NH, NH)] = w_stage[h, ...].astype(jnp.bfloat16)
        xcursor = [0]

        def next_x():
            i = xcursor[0]
            xcursor[0] = i + 1
            x_dma(i).wait()
            xv = x_stage[i % 2, ...].astype(jnp.bfloat16)
            if i + 2 < len(xtasks):
                x_dma(i + 2).start()
            return xv

        def dot_half(xv, col0):
            return lax.dot_general(
                xv, w_bf[:, pl.ds(col0, NH)],
                (((1,), (0,)), ((), ())),
                preferred_element_type=jnp.float32,
            )

        barrier_sem = pltpu.get_barrier_semaphore()
        for nbr in (left, right):
            pl.semaphore_signal(
                barrier_sem, inc=1,
                device_id=(nbr,), device_id_type=pl.DeviceIdType.MESH,
            )
        pl.semaphore_wait(barrier_sem, 2)

        def rs_rdma(d, s, k):
            return pltpu.make_async_remote_copy(
                src_ref=d["send"].at[s, rowsub(k), :],
                dst_ref=d["recv"].at[s, rowsub(k), :],
                send_sem=d["send_sems"].at[s * NS + k],
                recv_sem=d["recv_sems"].at[s * NS + k],
                device_id=(d["nbr"],),
                device_id_type=pl.DeviceIdType.MESH,
            )

        def ag_rdma(d, h, k):
            sc = d["ag_sc"](h)
            return pltpu.make_async_remote_copy(
                src_ref=d["ag"].at[sc, rowsub(k), :],
                dst_ref=d["ag"].at[sc, rowsub(k), :],
                send_sem=d["ag_send_sems"].at[h * NS + k],
                recv_sem=d["ag_recv_sems"].at[h * NS + k],
                device_id=(d["nbr"],),
                device_id_type=pl.DeviceIdType.MESH,
            )

        for k in range(NS):
            for d in DIRS:
                p = dot_half(next_x(), d["col0"])
                d["send"][0, rowsub(k), :] = p.astype(jnp.bfloat16)
                rs_rdma(d, 0, k).start()

        for k in range(NS):
            xv = next_x()
            for d in DIRS:
                p = dot_half(xv, d["col0"])
                rs_rdma(d, 0, k).wait_recv()
                acc = p + d["recv"][0, rowsub(k), :].astype(jnp.float32)
                d["send"][1, rowsub(k), :] = acc.astype(jnp.bfloat16)
                rs_rdma(d, 1, k).start()

        for k in range(NS):
            for d in DIRS:
                p = dot_half(next_x(), d["col0"])
                rs_rdma(d, 1, k).wait_recv()
                acc = p + d["recv"][1, rowsub(k), :].astype(jnp.float32)
                d["send"][2, rowsub(k), :] = acc.astype(jnp.bfloat16)
                rs_rdma(d, 2, k).start()

        amax_local = jnp.float32(0.0)
        for k in range(NS):
            xv = next_x()
            for d in DIRS:
                p = dot_half(xv, d["col0"])
                rs_rdma(d, W - 2, k).wait_recv()
                r = jnp.maximum(
                    p + d["recv"][W - 2, rowsub(k), :].astype(jnp.float32),
                    0.0)
                amax_local = jnp.maximum(amax_local, jnp.max(r))
                d["recv"][W - 2, rowsub(k), :] = r.astype(jnp.bfloat16)
        amax_buf[pl.ds(my, 1), ...] = jnp.full(
            (1, 8, 128), amax_local, jnp.float32)

        amax_sends = []
        for o in range(1, W):
            peer = (my + o) % W
            snd = pltpu.make_async_remote_copy(
                src_ref=amax_buf.at[my],
                dst_ref=amax_buf.at[my],
                send_sem=amax_send_sems.at[o - 1],
                recv_sem=amax_recv_sems.at[my],
                device_id=(peer,),
                device_id_type=pl.DeviceIdType.MESH,
            )
            snd.start()
            amax_sends.append(snd)
        for o in range(1, W):
            peer = (my + o) % W
            rcv = pltpu.make_async_remote_copy(
                src_ref=amax_buf.at[peer],
                dst_ref=amax_buf.at[peer],
                send_sem=amax_send_sems.at[0],
                recv_sem=amax_recv_sems.at[peer],
                device_id=(peer,),
                device_id_type=pl.DeviceIdType.MESH,
            )
            rcv.wait_recv()

        amax_g = jnp.max(amax_buf[...])
        scale = jnp.maximum(amax_g / 127.0, 1e-30)

        pending = [None, None]

        def stage_sub(values_f32, out_row, col0):
            t_slot = stage_sub.counter % 2
            stage_sub.counter += 1
            if pending[t_slot] is not None:
                pending[t_slot].wait()
            stage[t_slot, :, pl.ds(col0, NH)] = values_f32.astype(jnp.bfloat16)
            cp = pltpu.make_async_copy(
                stage.at[t_slot, :, pl.ds(col0, NH)],
                out_ref.at[pl.ds(out_row, SB), pl.ds(col0, NH)],
                stage_sems.at[t_slot],
            )
            cp.start()
            pending[t_slot] = cp
        stage_sub.counter = 0

        for k in range(NS):
            for d in DIRS:
                qf = jnp.clip(
                    jnp.round(
                        d["recv"][W - 2, rowsub(k), :].astype(jnp.float32)
                        / scale),
                    0.0, 127.0)
                d["ag"][pl.ds(my, 1), rowsub(k), :] = (
                    qf.astype(jnp.int8).reshape(1, SB, NH))
                ag_rdma(d, 0, k).start()
                stage_sub(qf * scale, my * MC + k * SB, d["col0"])

        for h in range(W - 1):
            for k in range(NS):
                for d in DIRS:
                    rc = d["ag_rc"](h)
                    ag_rdma(d, h, k).wait_recv()
                    if h < W - 2:
                        ag_rdma(d, h + 1, k).start()
                    stage_sub(
                        d["ag"][rc, rowsub(k), :].astype(jnp.float32) * scale,
                        rc * MC + k * SB, d["col0"])

        for s in range(W - 1):
            for k in range(NS):
                for d in DIRS:
                    rs_rdma(d, s, k).wait_send()
                    ag_rdma(d, s, k).wait_send()
        for snd in amax_sends:
            snd.wait_send()
        for cp in pending:
            if cp is not None:
                cp.wait()

    nsub = (W - 1) * NS
    return pl.pallas_call(
        body,
        out_shape=jax.ShapeDtypeStruct((M, N), jnp.bfloat16),
        in_specs=[
            pl.BlockSpec(memory_space=pl.ANY),
            pl.BlockSpec(memory_space=pl.ANY),
        ],
        out_specs=pl.BlockSpec(memory_space=pltpu.MemorySpace.HBM),
        scratch_shapes=[
            pltpu.VMEM((W - 1, MC, NH), jnp.bfloat16),
            pltpu.VMEM((W - 1, MC, NH), jnp.bfloat16),
            pltpu.VMEM((W - 1, MC, NH), jnp.bfloat16),
            pltpu.VMEM((W - 1, MC, NH), jnp.bfloat16),
            pltpu.VMEM((W, MC, NH), jnp.int8),
            pltpu.VMEM((W, MC, NH), jnp.int8),
            pltpu.VMEM((W, 8, 128), jnp.float32),
            pltpu.VMEM((2, SB, N), jnp.bfloat16),
            pltpu.VMEM((2, SB, Ks), jnp.float32),
            pltpu.VMEM((2, Ks, NH), jnp.float32),
            pltpu.VMEM((Ks, N), jnp.bfloat16),
            pltpu.SemaphoreType.DMA((nsub,)),
            pltpu.SemaphoreType.DMA((nsub,)),
            pltpu.SemaphoreType.DMA((nsub,)),
            pltpu.SemaphoreType.DMA((nsub,)),
            pltpu.SemaphoreType.DMA((nsub,)),
            pltpu.SemaphoreType.DMA((nsub,)),
            pltpu.SemaphoreType.DMA((nsub,)),
            pltpu.SemaphoreType.DMA((nsub,)),
            pltpu.SemaphoreType.DMA((W - 1,)),
            pltpu.SemaphoreType.DMA((W,)),
            pltpu.SemaphoreType.DMA((2,)),
            pltpu.SemaphoreType.DMA((2,)),
            pltpu.SemaphoreType.DMA((2,)),
        ],
        compiler_params=pltpu.CompilerParams(
            collective_id=0,
            vmem_limit_bytes=58 * 1024 * 1024,
        ),
    )(x, w_mat)


# device time: 132061 ns/iter; 2.1835x vs baseline; 1.0050x over previous
import jax
import jax.numpy as jnp
from jax import lax
from jax.experimental import pallas as pl
from jax.experimental.pallas import tpu as pltpu

W = 4
NS = 8


def kernel(x, w_mat):
    M, Ks = x.shape
    N = w_mat.shape[1]
    MC = M // W
    NH = N // 2
    SB = MC // NS

    def body(x_ref, w_ref, out_ref,
             rs_send_r, rs_send_l, rs_recv_r, rs_recv_l,
             ag_r, ag_l, amax_buf, stage, x_stage, w_stage, w_bf,
             rs_send_sems_r, rs_send_sems_l,
             rs_recv_sems_r, rs_recv_sems_l,
             ag_send_sems_r, ag_send_sems_l,
             ag_recv_sems_r, ag_recv_sems_l,
             amax_send_sems, amax_recv_sems,
             stage_sems, x_sems, w_sems):
        my = lax.axis_index("i")
        right = (my + 1) % W
        left = (my - 1) % W

        DIRS = (
            dict(nbr=right, col0=0, rs_chunk=lambda s: (my - 1 - s) % W,
                 send=rs_send_r, recv=rs_recv_r, ag=ag_r,
                 send_sems=rs_send_sems_r, recv_sems=rs_recv_sems_r,
                 ag_send_sems=ag_send_sems_r, ag_recv_sems=ag_recv_sems_r,
                 ag_sc=lambda h: (my - h) % W, ag_rc=lambda h: (my - h - 1) % W),
            dict(nbr=left, col0=NH, rs_chunk=lambda s: (my + 1 + s) % W,
                 send=rs_send_l, recv=rs_recv_l, ag=ag_l,
                 send_sems=rs_send_sems_l, recv_sems=rs_recv_sems_l,
                 ag_send_sems=ag_send_sems_l, ag_recv_sems=ag_recv_sems_l,
                 ag_sc=lambda h: (my + h) % W, ag_rc=lambda h: (my + h + 1) % W),
        )

        def rowsub(k):
            return pl.ds(k * SB, SB)

        xtasks = []
        for k in range(NS):
            xtasks.append(((my - 1) % W, k))
            xtasks.append(((my + 1) % W, k))
        for k in range(NS):
            xtasks.append(((my + 2) % W, k))
        for k in range(NS):
            xtasks.append(((my + 1) % W, k))
            xtasks.append(((my - 1) % W, k))
        for k in range(NS):
            xtasks.append((my, k))

        def x_dma(i):
            c, k = xtasks[i]
            return pltpu.make_async_copy(
                x_ref.at[pl.ds(c * MC + k * SB, SB), :],
                x_stage.at[i % 2],
                x_sems.at[i % 2],
            )

        w_dmas = [
            pltpu.make_async_copy(
                w_ref.at[:, pl.ds(h * NH, NH)], w_stage.at[h], w_sems.at[h])
            for h in range(2)
        ]
        for wd in w_dmas:
            wd.start()
        x_dma(0).start()
        x_dma(1).start()
        for h in range(2):
            w_dmas[h].wait()
            w_bf[:, pl.ds(h * NH, NH)] = w_stage[h, ...].astype(jnp.bfloat16)
        xcursor = [0]

        def next_x():
            i = xcursor[0]
            xcursor[0] = i + 1
            x_dma(i).wait()
            xv = x_stage[i % 2, ...].astype(jnp.bfloat16)
            if i + 2 < len(xtasks):
                x_dma(i + 2).start()
            return xv

        def dot_half(xv, col0):
            return lax.dot_general(
                xv, w_bf[:, pl.ds(col0, NH)],
                (((1,), (0,)), ((), ())),
                preferred_element_type=jnp.float32,
            )

        barrier_sem = pltpu.get_barrier_semaphore()
        for nbr in (left, right):
            pl.semaphore_signal(
                barrier_sem, inc=1,
                device_id=(nbr,), device_id_type=pl.DeviceIdType.MESH,
            )
        pl.semaphore_wait(barrier_sem, 2)

        def rs_rdma(d, s, k):
            return pltpu.make_async_remote_copy(
                src_ref=d["send"].at[s, rowsub(k), :],
                dst_ref=d["recv"].at[s, rowsub(k), :],
                send_sem=d["send_sems"].at[s * NS + k],
                recv_sem=d["recv_sems"].at[s * NS + k],
                device_id=(d["nbr"],),
                device_id_type=pl.DeviceIdType.MESH,
            )

        def ag_rdma(d, h, k):
            sc = d["ag_sc"](h)
            return pltpu.make_async_remote_copy(
                src_ref=d["ag"].at[sc, rowsub(k), :],
                dst_ref=d["ag"].at[sc, rowsub(k), :],
                send_sem=d["ag_send_sems"].at[h * NS + k],
                recv_sem=d["ag_recv_sems"].at[h * NS + k],
                device_id=(d["nbr"],),
                device_id_type=pl.DeviceIdType.MESH,
            )

        for k in range(NS):
            for d in DIRS:
                p = dot_half(next_x(), d["col0"])
                d["send"][0, rowsub(k), :] = p.astype(jnp.bfloat16)
                rs_rdma(d, 0, k).start()

        for k in range(NS):
            xv = next_x()
            for d in DIRS:
                p = dot_half(xv, d["col0"])
                rs_rdma(d, 0, k).wait_recv()
                acc = p + d["recv"][0, rowsub(k), :].astype(jnp.float32)
                d["send"][1, rowsub(k), :] = acc.astype(jnp.bfloat16)
                rs_rdma(d, 1, k).start()

        for k in range(NS):
            for d in DIRS:
                p = dot_half(next_x(), d["col0"])
                rs_rdma(d, 1, k).wait_recv()
                acc = p + d["recv"][1, rowsub(k), :].astype(jnp.float32)
                d["send"][2, rowsub(k), :] = acc.astype(jnp.bfloat16)
                rs_rdma(d, 2, k).start()

        amax_local = jnp.float32(0.0)
        for k in range(NS):
            xv = next_x()
            for d in DIRS:
                p = dot_half(xv, d["col0"])
                rs_rdma(d, W - 2, k).wait_recv()
                r = jnp.maximum(
                    p + d["recv"][W - 2, rowsub(k), :].astype(jnp.float32),
                    0.0)
                amax_local = jnp.maximum(amax_local, jnp.max(r))
                d["recv"][W - 2, rowsub(k), :] = r.astype(jnp.bfloat16)
        amax_buf[pl.ds(my, 1), ...] = jnp.full(
            (1, 8, 128), amax_local, jnp.float32)

        amax_sends = []
        for o in range(1, W):
            peer = (my + o) % W
            snd = pltpu.make_async_remote_copy(
                src_ref=amax_buf.at[my],
                dst_ref=amax_buf.at[my],
                send_sem=amax_send_sems.at[o - 1],
                recv_sem=amax_recv_sems.at[my],
                device_id=(peer,),
                device_id_type=pl.DeviceIdType.MESH,
            )
            snd.start()
            amax_sends.append(snd)
        for o in range(1, W):
            peer = (my + o) % W
            rcv = pltpu.make_async_remote_copy(
                src_ref=amax_buf.at[peer],
                dst_ref=amax_buf.at[peer],
                send_sem=amax_send_sems.at[0],
                recv_sem=amax_recv_sems.at[peer],
                device_id=(peer,),
                device_id_type=pl.DeviceIdType.MESH,
            )
            rcv.wait_recv()

        amax_g = jnp.max(amax_buf[...])
        scale = jnp.maximum(amax_g / 127.0, 1e-30)

        pending = [None, None]

        def stage_sub(values_f32, out_row, col0):
            t_slot = stage_sub.counter % 2
            stage_sub.counter += 1
            if pending[t_slot] is not None:
                pending[t_slot].wait()
            stage[t_slot, :, pl.ds(col0, NH)] = values_f32.astype(jnp.bfloat16)
            cp = pltpu.make_async_copy(
                stage.at[t_slot, :, pl.ds(col0, NH)],
                out_ref.at[pl.ds(out_row, SB), pl.ds(col0, NH)],
                stage_sems.at[t_slot],
            )
            cp.start()
            pending[t_slot] = cp
        stage_sub.counter = 0

        for k in range(NS):
            for d in DIRS:
                qf = jnp.clip(
                    jnp.round(
                        d["recv"][W - 2, rowsub(k), :].astype(jnp.float32)
                        / scale),
                    0.0, 127.0)
                d["ag"][pl.ds(my, 1), rowsub(k), :] = (
                    qf.astype(jnp.int8).reshape(1, SB, NH))
                ag_rdma(d, 0, k).start()
                stage_sub(qf * scale, my * MC + k * SB, d["col0"])

        for h in range(W - 1):
            for k in range(NS):
                for d in DIRS:
                    rc = d["ag_rc"](h)
                    ag_rdma(d, h, k).wait_recv()
                    if h < W - 2:
                        ag_rdma(d, h + 1, k).start()
                    stage_sub(
                        d["ag"][rc, rowsub(k), :].astype(jnp.float32) * scale,
                        rc * MC + k * SB, d["col0"])

        for s in range(W - 1):
            for k in range(NS):
                for d in DIRS:
                    rs_rdma(d, s, k).wait_send()
                    ag_rdma(d, s, k).wait_send()
        for snd in amax_sends:
            snd.wait_send()
        for cp in pending:
            if cp is not None:
                cp.wait()

    nsub = (W - 1) * NS
    return pl.pallas_call(
        body,
        out_shape=jax.ShapeDtypeStruct((M, N), jnp.bfloat16),
        in_specs=[
            pl.BlockSpec(memory_space=pl.ANY),
            pl.BlockSpec(memory_space=pl.ANY),
        ],
        out_specs=pl.BlockSpec(memory_space=pltpu.MemorySpace.HBM),
        scratch_shapes=[
            pltpu.VMEM((W - 1, MC, NH), jnp.bfloat16),
            pltpu.VMEM((W - 1, MC, NH), jnp.bfloat16),
            pltpu.VMEM((W - 1, MC, NH), jnp.bfloat16),
            pltpu.VMEM((W - 1, MC, NH), jnp.bfloat16),
            pltpu.VMEM((W, MC, NH), jnp.int8),
            pltpu.VMEM((W, MC, NH), jnp.int8),
            pltpu.VMEM((W, 8, 128), jnp.float32),
            pltpu.VMEM((2, SB, N), jnp.bfloat16),
            pltpu.VMEM((2, SB, Ks), jnp.float32),
            pltpu.VMEM((2, Ks, NH), jnp.float32),
            pltpu.VMEM((Ks, N), jnp.bfloat16),
            pltpu.SemaphoreType.DMA((nsub,)),
            pltpu.SemaphoreType.DMA((nsub,)),
            pltpu.SemaphoreType.DMA((nsub,)),
            pltpu.SemaphoreType.DMA((nsub,)),
            pltpu.SemaphoreType.DMA((nsub,)),
            pltpu.SemaphoreType.DMA((nsub,)),
            pltpu.SemaphoreType.DMA((nsub,)),
            pltpu.SemaphoreType.DMA((nsub,)),
            pltpu.SemaphoreType.DMA((W - 1,)),
            pltpu.SemaphoreType.DMA((W,)),
            pltpu.SemaphoreType.DMA((2,)),
            pltpu.SemaphoreType.DMA((2,)),
            pltpu.SemaphoreType.DMA((2,)),
        ],
        compiler_params=pltpu.CompilerParams(
            collective_id=0,
            vmem_limit_bytes=58 * 1024 * 1024,
        ),
    )(x, w_mat)
